# baseline (speedup 1.0000x reference)
import functools

import jax
import jax.numpy as jnp
from jax import lax
from jax.experimental import pallas as pl
from jax.experimental.pallas import tpu as pltpu
from jax.experimental.pallas import tpu_sc as plsc

B, H, S, D = 8, 16, 2048, 128
P = 1024
NBH = B * H
NC, NS = 2, 16
NW = NC * NS
JOBS = NBH // NW


def _body(kc, vc, kv, vv, idx, ko, vo, kv_v, vv_v, idx_v, csem, ssem):
    wid = lax.axis_index("s") * NC + lax.axis_index("c")

    copies = []
    for j in range(JOBS):
        bh = wid * JOBS + j
        copies.append(pltpu.async_copy(
            kc.at[pl.ds(bh * S, P), :], ko.at[pl.ds(bh * P, P), :], csem))
        copies.append(pltpu.async_copy(
            vc.at[pl.ds(bh * S, P), :], vo.at[pl.ds(bh * P, P), :], csem))

    pltpu.sync_copy(idx.at[wid], idx_v)
    pltpu.sync_copy(kv.at[pl.ds(wid * JOBS, JOBS), :], kv_v)
    pltpu.sync_copy(vv.at[pl.ds(wid * JOBS, JOBS), :], vv_v)

    for c in copies:
        c.wait()

    pltpu.async_copy(kv_v, ko.at[idx_v], ssem).wait()
    pltpu.async_copy(vv_v, vo.at[idx_v], ssem).wait()


@jax.jit
def _run(kc, vc, kv, vv, idx):
    mesh = plsc.VectorSubcoreMesh(core_axis_name="c", subcore_axis_name="s")
    f = functools.partial(
        pl.kernel,
        out_type=[jax.ShapeDtypeStruct((NBH * P, D), jnp.float32)] * 2,
        mesh=mesh,
        scratch_types=[
            pltpu.VMEM((JOBS, D), jnp.float32),
            pltpu.VMEM((JOBS, D), jnp.float32),
            pltpu.VMEM((JOBS,), jnp.int32),
            pltpu.SemaphoreType.DMA,
            pltpu.SemaphoreType.DMA,
        ],
    )(_body)
    return f(kc, vc, kv, vv, idx)


def kernel(k_cache, v_cache, k_val, v_val, input_pos):
    kc = k_cache.reshape(NBH * S, D)
    vc = v_cache.reshape(NBH * S, D)
    kv = k_val.reshape(NBH, D)
    vv = v_val.reshape(NBH, D)
    pos = jnp.asarray(input_pos, jnp.int32)
    idx = (jnp.arange(NBH, dtype=jnp.int32) * P + pos - 1).reshape(NW, JOBS)
    ko, vo = _run(kc, vc, kv, vv, idx)
    return ko.reshape(B, H, P, D), vo.reshape(B, H, P, D)


# speedup vs baseline: 35.0530x; 35.0530x over previous
import functools

import jax
import jax.numpy as jnp
from jax import lax
from jax.experimental import pallas as pl
from jax.experimental.pallas import tpu as pltpu
from jax.experimental.pallas import tpu_sc as plsc

B, H, S, D = 8, 16, 2048, 128
P = 1024
NBH = B * H
NC, NS = 2, 16
NW = NC * NS
JOBS = NBH // NW


CH = 128
NB = 4
LA = 2
CPJ = P // CH
NCH = 2 * JOBS * CPJ


def _chunk(refs, wid, i):
    kc, vc, ko, vo = refs
    job, c = divmod(i, CPJ)
    bh = wid * JOBS + job % JOBS
    src, dst = (kc, ko) if job < JOBS else (vc, vo)
    return (src.at[pl.ds(bh * S + c * CH, CH), :],
            dst.at[pl.ds(bh * P + c * CH, CH), :])


def _body(kc, vc, kv, vv, idx, ko, vo, bufs, kv_v, vv_v, idx_v,
          gsem, ssem, vsem):
    wid = lax.axis_index("s") * NC + lax.axis_index("c")
    refs = (kc, vc, ko, vo)

    pltpu.sync_copy(idx.at[wid], idx_v)
    pltpu.sync_copy(kv.at[pl.ds(wid * JOBS, JOBS), :], kv_v)
    pltpu.sync_copy(vv.at[pl.ds(wid * JOBS, JOBS), :], vv_v)

    gd = [None] * NCH
    sd = [None] * NCH
    for i in range(NCH + LA):
        if i < NCH:
            if i >= NB:
                sd[i - NB].wait()
            src, dst = _chunk(refs, wid, i)
            gd[i] = pltpu.async_copy(src, bufs.at[i % NB], gsem)
        j = i - LA
        if 0 <= j < NCH:
            gd[j].wait()
            _, dst = _chunk(refs, wid, j)
            sd[j] = pltpu.async_copy(bufs.at[j % NB], dst, ssem)
    for j in range(NCH - NB, NCH):
        sd[j].wait()

    pltpu.async_copy(kv_v, ko.at[idx_v], vsem).wait()
    pltpu.async_copy(vv_v, vo.at[idx_v], vsem).wait()


@jax.jit
def _run(kc, vc, kv, vv, idx):
    mesh = plsc.VectorSubcoreMesh(core_axis_name="c", subcore_axis_name="s")
    f = functools.partial(
        pl.kernel,
        out_type=[jax.ShapeDtypeStruct((NBH * P, D), jnp.float32)] * 2,
        mesh=mesh,
        scratch_types=[
            pltpu.VMEM((NB, CH, D), jnp.float32),
            pltpu.VMEM((JOBS, D), jnp.float32),
            pltpu.VMEM((JOBS, D), jnp.float32),
            pltpu.VMEM((JOBS,), jnp.int32),
            pltpu.SemaphoreType.DMA,
            pltpu.SemaphoreType.DMA,
            pltpu.SemaphoreType.DMA,
        ],
    )(_body)
    return f(kc, vc, kv, vv, idx)


def kernel(k_cache, v_cache, k_val, v_val, input_pos):
    kc = k_cache.reshape(NBH * S, D)
    vc = v_cache.reshape(NBH * S, D)
    kv = k_val.reshape(NBH, D)
    vv = v_val.reshape(NBH, D)
    pos = jnp.asarray(input_pos, jnp.int32)
    idx = (jnp.arange(NBH, dtype=jnp.int32) * P + pos - 1).reshape(NW, JOBS)
    ko, vo = _run(kc, vc, kv, vv, idx)
    return ko.reshape(B, H, P, D), vo.reshape(B, H, P, D)
